# baseline (speedup 1.0000x reference)
import jax
import jax.numpy as jnp
from jax.experimental import pallas as pl


def _moe_block(x_ref, u_ref, wg_ref, wexp_ref, bexp_ref, o_ref):
    C, NB, P = x_ref.shape[1], x_ref.shape[2], x_ref.shape[3]
    E = u_ref.shape[3]
    T = NB * P

    xb16 = x_ref[0].astype(jnp.bfloat16).reshape(C, T)

    g = jnp.dot(wg_ref[...].astype(jnp.bfloat16), xb16,
                preferred_element_type=jnp.float32)
    el = g[:E]
    nl = g[E:]
    nl_max = jnp.max(nl, axis=0, keepdims=True)
    nl_exp = jnp.exp(nl - nl_max)
    ut = u_ref[0].reshape(T, E).T
    noise = ut * (nl_exp / jnp.sum(nl_exp, axis=0, keepdims=True))
    logits = el + noise

    lmin = jnp.min(logits, axis=0, keepdims=True)
    eidx = jax.lax.broadcasted_iota(jnp.int32, logits.shape, 0)
    drop = jnp.max(jnp.where(logits == lmin, eidx, -1), axis=0, keepdims=True)
    keep = eidx != drop
    lmax = jnp.max(logits, axis=0, keepdims=True)
    ex = jnp.where(keep, jnp.exp(logits - lmax), 0.0)
    gates = ex / jnp.sum(ex, axis=0, keepdims=True)
    s = jnp.sum(gates, axis=0, keepdims=True)

    y = jnp.dot(wexp_ref[...].astype(jnp.bfloat16), xb16,
                preferred_element_type=jnp.float32)
    yo = (y + bexp_ref[...]) * s
    o_ref[0] = yo.reshape(yo.shape[0], NB, P)


def kernel(x, We, be, Wn, bn, Wexp, bexp, noise_uniform):
    B, C, N, P = x.shape
    E = We.shape[0]
    O = Wexp.shape[0]
    NB = 8

    u4 = noise_uniform.reshape(B, N, P, E)
    bexp2 = bexp.reshape(O, 1)
    Wg = jnp.concatenate([We, Wn], axis=0)

    out = pl.pallas_call(
        _moe_block,
        grid=(B, N // NB),
        in_specs=[
            pl.BlockSpec((1, C, NB, P), lambda b, i: (b, 0, i, 0)),
            pl.BlockSpec((1, NB, P, E), lambda b, i: (b, i, 0, 0)),
            pl.BlockSpec((2 * E, C), lambda b, i: (0, 0)),
            pl.BlockSpec((O, C), lambda b, i: (0, 0)),
            pl.BlockSpec((O, 1), lambda b, i: (0, 0)),
        ],
        out_specs=pl.BlockSpec((1, O, NB, P), lambda b, i: (b, 0, i, 0)),
        out_shape=jax.ShapeDtypeStruct((B, O, N, P), x.dtype),
    )(x, u4, Wg, Wexp, bexp2)
    return out


# speedup vs baseline: 1.0649x; 1.0649x over previous
import jax
import jax.numpy as jnp
from jax.experimental import pallas as pl


def _moe_block(x_ref, u_ref, wg_ref, wexp_ref, bexp_ref, o_ref):
    C, NB, P = x_ref.shape[1], x_ref.shape[2], x_ref.shape[3]
    E = u_ref.shape[3]
    T = NB * P

    xb16 = x_ref[0].astype(jnp.bfloat16).reshape(C, T)

    g = jnp.dot(wg_ref[...].astype(jnp.bfloat16), xb16,
                preferred_element_type=jnp.float32)
    el = g[:E]
    nl = g[E:]
    nl_max = jnp.max(nl, axis=0, keepdims=True)
    nl_exp = jnp.exp(nl - nl_max)
    ut = u_ref[0].reshape(T, E).T
    noise = ut * (nl_exp / jnp.sum(nl_exp, axis=0, keepdims=True))
    logits = el + noise

    lmin = jnp.min(logits, axis=0, keepdims=True)
    eidx = jax.lax.broadcasted_iota(jnp.int32, logits.shape, 0)
    drop = jnp.max(jnp.where(logits == lmin, eidx, -1), axis=0, keepdims=True)
    keep = eidx != drop
    lmax = jnp.max(logits, axis=0, keepdims=True)
    ex = jnp.where(keep, jnp.exp(logits - lmax), 0.0)
    gates = ex / jnp.sum(ex, axis=0, keepdims=True)
    s = jnp.sum(gates, axis=0, keepdims=True)

    y = jnp.dot(wexp_ref[...].astype(jnp.bfloat16), xb16,
                preferred_element_type=jnp.float32)
    yo = (y + bexp_ref[...]) * s
    o_ref[0] = yo.reshape(yo.shape[0], NB, P)


def kernel(x, We, be, Wn, bn, Wexp, bexp, noise_uniform):
    B, C, N, P = x.shape
    E = We.shape[0]
    O = Wexp.shape[0]
    NB = 16

    u4 = noise_uniform.reshape(B, N, P, E)
    bexp2 = bexp.reshape(O, 1)
    Wg = jnp.concatenate([We, Wn], axis=0)

    out = pl.pallas_call(
        _moe_block,
        grid=(B, N // NB),
        in_specs=[
            pl.BlockSpec((1, C, NB, P), lambda b, i: (b, 0, i, 0)),
            pl.BlockSpec((1, NB, P, E), lambda b, i: (b, i, 0, 0)),
            pl.BlockSpec((2 * E, C), lambda b, i: (0, 0)),
            pl.BlockSpec((O, C), lambda b, i: (0, 0)),
            pl.BlockSpec((O, 1), lambda b, i: (0, 0)),
        ],
        out_specs=pl.BlockSpec((1, O, NB, P), lambda b, i: (b, 0, i, 0)),
        out_shape=jax.ShapeDtypeStruct((B, O, N, P), x.dtype),
    )(x, u4, Wg, Wexp, bexp2)
    return out
